# Initial kernel scaffold; baseline (speedup 1.0000x reference)
import jax
import jax.numpy as jnp
from jax.experimental import pallas as pl


def kernel(x, W0, as0, ad0, b0, W1, as1, ad1, b1, W2, as2, ad2, b2, fc1_w, fc1_b, fc2_w, fc2_b, bn1_g, bn1_b, bn2_g, bn2_b, out_w, out_b, edge_index, batch):
    raise NotImplementedError("write your pallas kernel here")


# speedup vs baseline: 61.9095x; 61.9095x over previous
import functools

import jax
import jax.numpy as jnp
from jax import lax
from jax.experimental import pallas as pl
from jax.experimental.pallas import tpu as pltpu
from jax.experimental.pallas import tpu_sc as plsc

N = 10000
E = 320000
IN = 128
HID = 16
HEADS = 8
LIN = 64
G = 64

NPAD = 10240
BLK = 128
NTILES = 32
NB = 81
EPAD = NTILES * NB * BLK
ROWS_PER_TILE = NPAD // 16
TBLK = 512


def _make_edge_kernel(D):
  nch = D // 16
  mesh = plsc.VectorSubcoreMesh(core_axis_name="c", subcore_axis_name="s")
  out_type = (
      jax.ShapeDtypeStruct((2, NPAD, D), jnp.float32),
      jax.ShapeDtypeStruct((2, NPAD, 16), jnp.float32),
  )
  scratch = [
      pltpu.VMEM_SHARED((NPAD, D), jnp.float32),
      pltpu.VMEM_SHARED((NPAD, 16), jnp.float32),
      pltpu.VMEM((BLK, D), jnp.float32),
      pltpu.VMEM((BLK, 16), jnp.float32),
      pltpu.VMEM((BLK, 16), jnp.float32),
      pltpu.VMEM((BLK, 16), jnp.float32),
      pltpu.VMEM((BLK,), jnp.int32),
      pltpu.VMEM((BLK,), jnp.int32),
      pltpu.VMEM((16,), jnp.float32),
      pltpu.SemaphoreType.DMA,
      pltpu.SemaphoreType.DMA,
      pltpu.SemaphoreType.DMA,
  ]

  @functools.partial(
      pl.kernel, out_type=out_type, mesh=mesh, scratch_types=scratch,
      compiler_params=pltpu.CompilerParams(use_tc_tiling_on_sc=False))
  def k(xl_hbm, tabs_hbm, tabd_hbm, m_hbm, src_hbm, dst_hbm,
        acc_out, den_out,
        acc, den, xbuf, sbuf, dbuf, wbuf, isrc, idst, mbuf,
        sem0, sem1, sem2):
    c = lax.axis_index("c")
    s = lax.axis_index("s")
    tf = c * 16 + s
    zero16 = jnp.zeros((16,), jnp.float32)
    pltpu.sync_copy(m_hbm, mbuf)
    mv = mbuf[...]

    def zrow(r, carry):
      for j in range(nch):
        xbuf[r, pl.ds(j * 16, 16)] = zero16
      wbuf[r, pl.ds(0, 16)] = zero16
      return carry
    lax.fori_loop(0, BLK, zrow, 0)
    for kk in range(ROWS_PER_TILE // BLK):
      base = s * ROWS_PER_TILE + kk * BLK
      pltpu.sync_copy(xbuf, acc.at[pl.ds(base, BLK)])
      pltpu.sync_copy(wbuf, den.at[pl.ds(base, BLK)])
    plsc.subcore_barrier()

    def block_body(b, carry):
      row = tf * NB + b
      pltpu.sync_copy(src_hbm.at[row], isrc)
      pltpu.sync_copy(dst_hbm.at[row], idst)
      cp0 = pltpu.async_copy(xl_hbm.at[isrc], xbuf, sem0)
      cp1 = pltpu.async_copy(tabs_hbm.at[isrc], sbuf, sem1)
      cp2 = pltpu.async_copy(tabd_hbm.at[idst], dbuf, sem2)
      cp1.wait()
      cp2.wait()
      cp0.wait()

      def edge_body(e, ecarry):
        a = sbuf[e, :] + dbuf[e, :]
        a = jnp.maximum(a, a * 0.2)
        w = jnp.exp(a - mv)
        wbuf[e, :] = w
        for j in range(nch):
          m = lax.gather(
              w, jnp.full((16, 1), j, jnp.int32),
              lax.GatherDimensionNumbers(offset_dims=(),
                                         collapsed_slice_dims=(0,),
                                         start_index_map=(0,)),
              (1,), mode=lax.GatherScatterMode.PROMISE_IN_BOUNDS)
          xbuf[e, pl.ds(j * 16, 16)] = xbuf[e, pl.ds(j * 16, 16)] * m
        return ecarry
      lax.fori_loop(0, BLK, edge_body, 0)

      pltpu.sync_copy(xbuf, acc.at[idst], add=True)
      pltpu.sync_copy(wbuf, den.at[idst], add=True)
      return carry
    lax.fori_loop(0, NB, block_body, 0)

    plsc.subcore_barrier()
    base = s * ROWS_PER_TILE
    pltpu.sync_copy(acc.at[pl.ds(base, ROWS_PER_TILE)],
                    acc_out.at[c, pl.ds(base, ROWS_PER_TILE)])
    pltpu.sync_copy(den.at[pl.ds(base, ROWS_PER_TILE)],
                    den_out.at[c, pl.ds(base, ROWS_PER_TILE)])

  return k


_edge128 = _make_edge_kernel(IN)
_edge16 = _make_edge_kernel(HID)


def _t0(xpad, w0t, asm, adm):
  def body(x_ref, w_ref, as_ref, ad_ref, xl_ref, ts_ref, td_ref):
    xl = jnp.dot(x_ref[...], w_ref[...], preferred_element_type=jnp.float32, precision=lax.Precision.HIGHEST)
    xl_ref[...] = xl
    ts_ref[...] = jnp.dot(xl, as_ref[...], preferred_element_type=jnp.float32, precision=lax.Precision.HIGHEST)
    td_ref[...] = jnp.dot(xl, ad_ref[...], preferred_element_type=jnp.float32, precision=lax.Precision.HIGHEST)
  return pl.pallas_call(
      body,
      grid=(NPAD // TBLK,),
      in_specs=[pl.BlockSpec((TBLK, IN), lambda i: (i, 0)),
                pl.BlockSpec((IN, IN), lambda i: (0, 0)),
                pl.BlockSpec((IN, 16), lambda i: (0, 0)),
                pl.BlockSpec((IN, 16), lambda i: (0, 0))],
      out_specs=[pl.BlockSpec((TBLK, IN), lambda i: (i, 0)),
                 pl.BlockSpec((TBLK, 16), lambda i: (i, 0)),
                 pl.BlockSpec((TBLK, 16), lambda i: (i, 0))],
      out_shape=[jax.ShapeDtypeStruct((NPAD, IN), jnp.float32),
                 jax.ShapeDtypeStruct((NPAD, 16), jnp.float32),
                 jax.ShapeDtypeStruct((NPAD, 16), jnp.float32)],
  )(xpad, w0t, asm, adm)


def _tmid(acc, den, bias, rm, wt, asm, adm, d_out):
  def body(acc_ref, den_ref, b_ref, r_ref, w_ref, as_ref, ad_ref,
           xl_ref, ts_ref, td_ref):
    sacc = acc_ref[0] + acc_ref[1]
    sden = den_ref[0] + den_ref[1]
    div = jnp.dot(sden, r_ref[...], preferred_element_type=jnp.float32, precision=lax.Precision.HIGHEST) + 1e-16
    h = sacc / div + b_ref[...]
    h = jnp.where(h > 0, h, jnp.exp(h) - 1.0)
    xl = jnp.dot(h, w_ref[...], preferred_element_type=jnp.float32, precision=lax.Precision.HIGHEST)
    xl_ref[...] = xl
    ts_ref[...] = jnp.dot(xl, as_ref[...], preferred_element_type=jnp.float32, precision=lax.Precision.HIGHEST)
    td_ref[...] = jnp.dot(xl, ad_ref[...], preferred_element_type=jnp.float32, precision=lax.Precision.HIGHEST)
  return pl.pallas_call(
      body,
      grid=(NPAD // TBLK,),
      in_specs=[pl.BlockSpec((2, TBLK, IN), lambda i: (0, i, 0)),
                pl.BlockSpec((2, TBLK, 16), lambda i: (0, i, 0)),
                pl.BlockSpec((1, IN), lambda i: (0, 0)),
                pl.BlockSpec((16, IN), lambda i: (0, 0)),
                pl.BlockSpec((IN, d_out), lambda i: (0, 0)),
                pl.BlockSpec((d_out, 16), lambda i: (0, 0)),
                pl.BlockSpec((d_out, 16), lambda i: (0, 0))],
      out_specs=[pl.BlockSpec((TBLK, d_out), lambda i: (i, 0)),
                 pl.BlockSpec((TBLK, 16), lambda i: (i, 0)),
                 pl.BlockSpec((TBLK, 16), lambda i: (i, 0))],
      out_shape=[jax.ShapeDtypeStruct((NPAD, d_out), jnp.float32),
                 jax.ShapeDtypeStruct((NPAD, 16), jnp.float32),
                 jax.ShapeDtypeStruct((NPAD, 16), jnp.float32)],
  )(acc, den, bias, rm, wt, asm, adm)


def _t3(acc2, den2, b2, r2, batch_row):
  def body(acc_ref, den_ref, b_ref, r_ref, bt_ref, ps_ref, cnt_ref):
    i = pl.program_id(0)
    sacc = acc_ref[0] + acc_ref[1]
    sden = den_ref[0] + den_ref[1]
    div = jnp.dot(sden, r_ref[...], preferred_element_type=jnp.float32, precision=lax.Precision.HIGHEST) + 1e-16
    h = sacc / div + b_ref[...]
    h = jnp.where(h > 0, h, jnp.exp(h) - 1.0)
    oh = (lax.broadcasted_iota(jnp.int32, (G, TBLK), 0)
          == bt_ref[...]).astype(jnp.float32)
    ps = jnp.dot(oh, h, preferred_element_type=jnp.float32, precision=lax.Precision.HIGHEST)
    cn = jnp.broadcast_to(jnp.sum(oh, axis=1, keepdims=True), (G, 16))

    @pl.when(i == 0)
    def _init():
      ps_ref[...] = jnp.zeros_like(ps_ref)
      cnt_ref[...] = jnp.zeros_like(cnt_ref)
    ps_ref[...] += ps
    cnt_ref[...] += cn
  return pl.pallas_call(
      body,
      grid=(NPAD // TBLK,),
      in_specs=[pl.BlockSpec((2, TBLK, 16), lambda i: (0, i, 0)),
                pl.BlockSpec((2, TBLK, 16), lambda i: (0, i, 0)),
                pl.BlockSpec((1, 16), lambda i: (0, 0)),
                pl.BlockSpec((16, 16), lambda i: (0, 0)),
                pl.BlockSpec((1, TBLK), lambda i: (0, i))],
      out_specs=[pl.BlockSpec((G, 16), lambda i: (0, 0)),
                 pl.BlockSpec((G, 16), lambda i: (0, 0))],
      out_shape=[jax.ShapeDtypeStruct((G, 16), jnp.float32),
                 jax.ShapeDtypeStruct((G, 16), jnp.float32)],
  )(acc2, den2, b2, r2, batch_row)


def _t4(ps, cnt, w1t, b1r, w2t, b2r, g1, be1, g2, be2, wot, bor):
  rsq = float((1.0 + 1e-5) ** -0.5)

  def body(ps_ref, cnt_ref, w1, b1, w2, b2, g1r, e1r, g2r, e2r, wo, bo,
           out_ref):
    pool = ps_ref[...] / jnp.maximum(cnt_ref[...], 1.0)
    h = jnp.maximum(
        jnp.dot(pool, w1[...], preferred_element_type=jnp.float32, precision=lax.Precision.HIGHEST) + b1[...],
        0.0)
    h = h * (g1r[...] * rsq) + e1r[...]
    h = jnp.maximum(
        jnp.dot(h, w2[...], preferred_element_type=jnp.float32, precision=lax.Precision.HIGHEST) + b2[...],
        0.0)
    h = h * (g2r[...] * rsq) + e2r[...]
    out_ref[...] = (
        jnp.dot(h, wo[...], preferred_element_type=jnp.float32, precision=lax.Precision.HIGHEST) + bo[...])
  return pl.pallas_call(
      body,
      out_shape=jax.ShapeDtypeStruct((G, 1), jnp.float32),
  )(ps, cnt, w1t, b1r, w2t, b2r, g1, be1, g2, be2, wot, bor)


def _amat(a_flat):
  dm = a_flat.shape[0]
  cidx = jnp.arange(dm)[:, None] // HID
  hidx = jnp.arange(16)[None, :]
  return jnp.where(cidx == hidx, a_flat[:, None], 0.0).astype(jnp.float32)


def _mvec(ts, td):
  b = jnp.max(ts, axis=0) + jnp.max(td, axis=0)
  return jnp.maximum(b, 0.2 * b)


def kernel(x, W0, as0, ad0, b0, W1, as1, ad1, b1, W2, as2, ad2, b2,
           fc1_w, fc1_b, fc2_w, fc2_b, bn1_g, bn1_b, bn2_g, bn2_b,
           out_w, out_b, edge_index, batch):
  f32 = jnp.float32
  sl = jnp.arange(N, dtype=edge_index.dtype)
  src = jnp.concatenate([edge_index[0], sl,
                         jnp.zeros((EPAD - E - N,), edge_index.dtype)])
  dst = jnp.concatenate([edge_index[1], sl,
                         jnp.full((EPAD - E - N,), NPAD - 1,
                                  edge_index.dtype)])
  src2d = src.reshape(NTILES * NB, BLK)
  dst2d = dst.reshape(NTILES * NB, BLK)

  rm = (jnp.arange(16)[:, None]
        == (jnp.arange(IN)[None, :] // HID)).astype(f32)
  r2 = (jnp.arange(16)[:, None]
        == (jnp.arange(16)[None, :] // HID)).astype(f32)

  xpad = jnp.pad(x, ((0, NPAD - N), (0, 0)))

  xl0, ts0, td0 = _t0(xpad, W0.T, _amat(as0.reshape(-1)),
                      _amat(ad0.reshape(-1)))
  acc0, den0 = _edge128(xl0, ts0, td0, _mvec(ts0, td0), src2d, dst2d)

  xl1, ts1, td1 = _tmid(acc0, den0, b0.reshape(1, -1), rm, W1.T,
                        _amat(as1.reshape(-1)), _amat(ad1.reshape(-1)), IN)
  acc1, den1 = _edge128(xl1, ts1, td1, _mvec(ts1, td1), src2d, dst2d)

  xl2, ts2, td2 = _tmid(acc1, den1, b1.reshape(1, -1), rm, W2.T,
                        _amat(as2.reshape(-1)), _amat(ad2.reshape(-1)), HID)
  acc2, den2 = _edge16(xl2, ts2, td2, _mvec(ts2, td2), src2d, dst2d)

  batch_row = jnp.pad(batch, (0, NPAD - N),
                      constant_values=G).reshape(1, NPAD).astype(jnp.int32)
  ps, cnt = _t3(acc2, den2, b2.reshape(1, -1), r2, batch_row)

  return _t4(ps, cnt, fc1_w.T, fc1_b.reshape(1, -1), fc2_w.T,
             fc2_b.reshape(1, -1), bn1_g.reshape(1, -1),
             bn1_b.reshape(1, -1), bn2_g.reshape(1, -1),
             bn2_b.reshape(1, -1), out_w.T, out_b.reshape(1, -1))


# speedup vs baseline: 106.2004x; 1.7154x over previous
import functools

import jax
import jax.numpy as jnp
from jax import lax
from jax.experimental import pallas as pl
from jax.experimental.pallas import tpu as pltpu
from jax.experimental.pallas import tpu_sc as plsc

N = 10000
E = 320000
IN = 128
HID = 16
HEADS = 8
LIN = 64
G = 64

NPAD = 10112
BLK = 48
NTILES = 32
NB = 216
NSEG = 4
HB = NB // NSEG
EPAD = NTILES * NB * BLK
ROWS_PER_TILE = NPAD // 16
TBLK = 632


def _make_edge_kernel(D):
  nch = D // 16
  mesh = plsc.VectorSubcoreMesh(core_axis_name="c", subcore_axis_name="s")
  out_type = (
      jax.ShapeDtypeStruct((2, NPAD, D), jnp.float32),
      jax.ShapeDtypeStruct((2, NPAD, 16), jnp.float32),
  )
  scratch = [
      pltpu.VMEM_SHARED((NPAD, D), jnp.float32),
      pltpu.VMEM_SHARED((NPAD, 16), jnp.float32),
      pltpu.VMEM((BLK, D), jnp.float32),
      pltpu.VMEM((BLK, D), jnp.float32),
      pltpu.VMEM((BLK, D), jnp.float32),
      pltpu.VMEM((BLK, 16), jnp.float32),
      pltpu.VMEM((BLK, 16), jnp.float32),
      pltpu.VMEM((BLK, 16), jnp.float32),
      pltpu.VMEM((BLK, 16), jnp.float32),
      pltpu.VMEM((BLK, 16), jnp.float32),
      pltpu.VMEM((BLK, 16), jnp.float32),
      pltpu.VMEM((BLK, 16), jnp.float32),
      pltpu.VMEM((BLK, 16), jnp.float32),
      pltpu.VMEM((BLK, 16), jnp.float32),
      pltpu.VMEM((HB, BLK), jnp.int32),
      pltpu.VMEM((HB, BLK), jnp.int32),
      pltpu.VMEM((16,), jnp.float32),
      pltpu.SemaphoreType.DMA,
      pltpu.SemaphoreType.DMA,
      pltpu.SemaphoreType.DMA,
      pltpu.SemaphoreType.DMA,
      pltpu.SemaphoreType.DMA,
      pltpu.SemaphoreType.DMA,
  ]

  @functools.partial(
      pl.kernel, out_type=out_type, mesh=mesh, scratch_types=scratch,
      compiler_params=pltpu.CompilerParams(use_tc_tiling_on_sc=False))
  def k(xl_hbm, tabs_hbm, tabd_hbm, m_hbm, src_hbm, dst_hbm,
        acc_out, den_out,
        acc, den, xb0, xb1, xb2, sb0, sb1, sb2, db0, db1, db2,
        wb0, wb1, wb2, sidx, didx, mbuf,
        semg0, semg1, semg2, sems0, sems1, sems2):
    c = lax.axis_index("c")
    s = lax.axis_index("s")
    tf = c * 16 + s
    zero16 = jnp.zeros((16,), jnp.float32)
    pltpu.sync_copy(m_hbm, mbuf)
    mv = mbuf[...]

    xbufs = (xb0, xb1, xb2)
    sbufs = (sb0, sb1, sb2)
    dbufs = (db0, db1, db2)
    wbufs = (wb0, wb1, wb2)
    gsems = (semg0, semg1, semg2)
    ssems = (sems0, sems1, sems2)

    @plsc.parallel_loop(0, BLK, 1, unroll=2)
    def _zero(r):
      for j in range(nch):
        xb0[r, pl.ds(j * 16, 16)] = zero16
        xb2[r, pl.ds(j * 16, 16)] = zero16
      wb0[r, pl.ds(0, 16)] = zero16
      wb2[r, pl.ds(0, 16)] = zero16
    nfull = ROWS_PER_TILE // BLK
    rem = ROWS_PER_TILE - nfull * BLK
    for kk in range(nfull):
      base = s * ROWS_PER_TILE + kk * BLK
      pltpu.sync_copy(xb0, acc.at[pl.ds(base, BLK)])
      pltpu.sync_copy(wb0, den.at[pl.ds(base, BLK)])
    if rem:
      base = s * ROWS_PER_TILE + nfull * BLK
      pltpu.sync_copy(xb0.at[pl.ds(0, rem)], acc.at[pl.ds(base, rem)])
      pltpu.sync_copy(wb0.at[pl.ds(0, rem)], den.at[pl.ds(base, rem)])
    plsc.subcore_barrier()

    def issue_gather(p, b):
      pltpu.async_copy(xl_hbm.at[sidx.at[b]], xbufs[p], gsems[p])
      pltpu.async_copy(tabs_hbm.at[sidx.at[b]], sbufs[p], gsems[p])
      pltpu.async_copy(tabd_hbm.at[didx.at[b]], dbufs[p], gsems[p])

    def gather_descs(p, b):
      d0 = pltpu.make_async_copy(xl_hbm.at[sidx.at[b]], xbufs[p], gsems[p])
      d1 = pltpu.make_async_copy(tabs_hbm.at[sidx.at[b]], sbufs[p], gsems[p])
      d2 = pltpu.make_async_copy(tabd_hbm.at[didx.at[b]], dbufs[p], gsems[p])
      return d0, d1, d2

    def issue_scatter(p, b):
      pltpu.async_copy(xbufs[p], acc.at[didx.at[b]], ssems[p], add=True)
      pltpu.async_copy(wbufs[p], den.at[didx.at[b]], ssems[p], add=True)

    def scatter_descs(p, b):
      d0 = pltpu.make_async_copy(xbufs[p], acc.at[didx.at[b]], ssems[p])
      d1 = pltpu.make_async_copy(wbufs[p], den.at[didx.at[b]], ssems[p])
      return d0, d1

    def compute(p):
      xb, sb, db, wb = xbufs[p], sbufs[p], dbufs[p], wbufs[p]

      @plsc.parallel_loop(0, BLK, 1, unroll=4)
      def _edges(e):
        a = sb[e, :] + db[e, :]
        a = jnp.maximum(a, a * 0.2)
        w = jnp.exp(a - mv)
        wb[e, :] = w
        for j in range(nch):
          m = lax.gather(
              w, jnp.full((16, 1), j, jnp.int32),
              lax.GatherDimensionNumbers(offset_dims=(),
                                         collapsed_slice_dims=(0,),
                                         start_index_map=(0,)),
              (1,), mode=lax.GatherScatterMode.PROMISE_IN_BOUNDS)
          xb[e, pl.ds(j * 16, 16)] = xb[e, pl.ds(j * 16, 16)] * m

    def triple_body(t, carry):
      b0 = t * 3
      for i in range(3):
        p = i
        q = (i - 1) % 3
        b = b0 + i
        for dd in gather_descs(p, b):
          dd.wait()
        compute(p)
        issue_scatter(p, b)
        for dd in scatter_descs(q, jnp.maximum(b - 1, 0)):
          dd.wait()
        issue_gather(q, jnp.minimum(b + 2, HB - 1))
      return carry

    for half in range(NSEG):
      pltpu.sync_copy(src_hbm.at[pl.ds(tf * NB + half * HB, HB)], sidx)
      pltpu.sync_copy(dst_hbm.at[pl.ds(tf * NB + half * HB, HB)], didx)
      if half:
        @plsc.parallel_loop(0, BLK, 1, unroll=2)
        def _rezero(r):
          for j in range(nch):
            xb2[r, pl.ds(j * 16, 16)] = zero16
          wb2[r, pl.ds(0, 16)] = zero16
      issue_scatter(2, 0)
      issue_gather(0, 0)
      issue_gather(1, 1)
      lax.fori_loop(0, HB // 3, triple_body, 0)
      for p in (0, 1):
        for dd in gather_descs(p, HB - 1):
          dd.wait()
      for dd in scatter_descs(2, HB - 1):
        dd.wait()

    plsc.subcore_barrier()
    base = s * ROWS_PER_TILE
    pltpu.sync_copy(acc.at[pl.ds(base, ROWS_PER_TILE)],
                    acc_out.at[c, pl.ds(base, ROWS_PER_TILE)])
    pltpu.sync_copy(den.at[pl.ds(base, ROWS_PER_TILE)],
                    den_out.at[c, pl.ds(base, ROWS_PER_TILE)])

  return k


_edge128 = _make_edge_kernel(IN)
_edge16 = _make_edge_kernel(HID)


def _t0(xpad, w0t, asm, adm):
  def body(x_ref, w_ref, as_ref, ad_ref, xl_ref, ts_ref, td_ref):
    xl = jnp.dot(x_ref[...], w_ref[...], preferred_element_type=jnp.float32, precision=lax.Precision.HIGHEST)
    xl_ref[...] = xl
    ts_ref[...] = jnp.dot(xl, as_ref[...], preferred_element_type=jnp.float32, precision=lax.Precision.HIGHEST)
    td_ref[...] = jnp.dot(xl, ad_ref[...], preferred_element_type=jnp.float32, precision=lax.Precision.HIGHEST)
  return pl.pallas_call(
      body,
      grid=(NPAD // TBLK,),
      in_specs=[pl.BlockSpec((TBLK, IN), lambda i: (i, 0)),
                pl.BlockSpec((IN, IN), lambda i: (0, 0)),
                pl.BlockSpec((IN, 16), lambda i: (0, 0)),
                pl.BlockSpec((IN, 16), lambda i: (0, 0))],
      out_specs=[pl.BlockSpec((TBLK, IN), lambda i: (i, 0)),
                 pl.BlockSpec((TBLK, 16), lambda i: (i, 0)),
                 pl.BlockSpec((TBLK, 16), lambda i: (i, 0))],
      out_shape=[jax.ShapeDtypeStruct((NPAD, IN), jnp.float32),
                 jax.ShapeDtypeStruct((NPAD, 16), jnp.float32),
                 jax.ShapeDtypeStruct((NPAD, 16), jnp.float32)],
  )(xpad, w0t, asm, adm)


def _tmid(acc, den, bias, rm, wt, asm, adm, d_out):
  def body(acc_ref, den_ref, b_ref, r_ref, w_ref, as_ref, ad_ref,
           xl_ref, ts_ref, td_ref):
    sacc = acc_ref[0] + acc_ref[1]
    sden = den_ref[0] + den_ref[1]
    div = jnp.dot(sden, r_ref[...], preferred_element_type=jnp.float32, precision=lax.Precision.HIGHEST) + 1e-16
    h = sacc / div + b_ref[...]
    h = jnp.where(h > 0, h, jnp.exp(h) - 1.0)
    xl = jnp.dot(h, w_ref[...], preferred_element_type=jnp.float32, precision=lax.Precision.HIGHEST)
    xl_ref[...] = xl
    ts_ref[...] = jnp.dot(xl, as_ref[...], preferred_element_type=jnp.float32, precision=lax.Precision.HIGHEST)
    td_ref[...] = jnp.dot(xl, ad_ref[...], preferred_element_type=jnp.float32, precision=lax.Precision.HIGHEST)
  return pl.pallas_call(
      body,
      grid=(NPAD // TBLK,),
      in_specs=[pl.BlockSpec((2, TBLK, IN), lambda i: (0, i, 0)),
                pl.BlockSpec((2, TBLK, 16), lambda i: (0, i, 0)),
                pl.BlockSpec((1, IN), lambda i: (0, 0)),
                pl.BlockSpec((16, IN), lambda i: (0, 0)),
                pl.BlockSpec((IN, d_out), lambda i: (0, 0)),
                pl.BlockSpec((d_out, 16), lambda i: (0, 0)),
                pl.BlockSpec((d_out, 16), lambda i: (0, 0))],
      out_specs=[pl.BlockSpec((TBLK, d_out), lambda i: (i, 0)),
                 pl.BlockSpec((TBLK, 16), lambda i: (i, 0)),
                 pl.BlockSpec((TBLK, 16), lambda i: (i, 0))],
      out_shape=[jax.ShapeDtypeStruct((NPAD, d_out), jnp.float32),
                 jax.ShapeDtypeStruct((NPAD, 16), jnp.float32),
                 jax.ShapeDtypeStruct((NPAD, 16), jnp.float32)],
  )(acc, den, bias, rm, wt, asm, adm)


def _t3(acc2, den2, b2, r2, batch_row):
  def body(acc_ref, den_ref, b_ref, r_ref, bt_ref, ps_ref, cnt_ref):
    i = pl.program_id(0)
    sacc = acc_ref[0] + acc_ref[1]
    sden = den_ref[0] + den_ref[1]
    div = jnp.dot(sden, r_ref[...], preferred_element_type=jnp.float32, precision=lax.Precision.HIGHEST) + 1e-16
    h = sacc / div + b_ref[...]
    h = jnp.where(h > 0, h, jnp.exp(h) - 1.0)
    oh = (lax.broadcasted_iota(jnp.int32, (TBLK, G), 1)
          == bt_ref[...]).astype(jnp.float32)
    ps = lax.dot_general(oh, h, (((0,), (0,)), ((), ())),
                         preferred_element_type=jnp.float32,
                         precision=lax.Precision.HIGHEST)
    cn = jnp.broadcast_to(jnp.sum(oh, axis=0)[:, None], (G, 16))

    @pl.when(i == 0)
    def _init():
      ps_ref[...] = jnp.zeros_like(ps_ref)
      cnt_ref[...] = jnp.zeros_like(cnt_ref)
    ps_ref[...] += ps
    cnt_ref[...] += cn
  return pl.pallas_call(
      body,
      grid=(NPAD // TBLK,),
      in_specs=[pl.BlockSpec((2, TBLK, 16), lambda i: (0, i, 0)),
                pl.BlockSpec((2, TBLK, 16), lambda i: (0, i, 0)),
                pl.BlockSpec((1, 16), lambda i: (0, 0)),
                pl.BlockSpec((16, 16), lambda i: (0, 0)),
                pl.BlockSpec((TBLK, 1), lambda i: (i, 0))],
      out_specs=[pl.BlockSpec((G, 16), lambda i: (0, 0)),
                 pl.BlockSpec((G, 16), lambda i: (0, 0))],
      out_shape=[jax.ShapeDtypeStruct((G, 16), jnp.float32),
                 jax.ShapeDtypeStruct((G, 16), jnp.float32)],
  )(acc2, den2, b2, r2, batch_row)


def _t4(ps, cnt, w1t, b1r, w2t, b2r, g1, be1, g2, be2, wot, bor):
  rsq = float((1.0 + 1e-5) ** -0.5)

  def body(ps_ref, cnt_ref, w1, b1, w2, b2, g1r, e1r, g2r, e2r, wo, bo,
           out_ref):
    pool = ps_ref[...] / jnp.maximum(cnt_ref[...], 1.0)
    h = jnp.maximum(
        jnp.dot(pool, w1[...], preferred_element_type=jnp.float32, precision=lax.Precision.HIGHEST) + b1[...],
        0.0)
    h = h * (g1r[...] * rsq) + e1r[...]
    h = jnp.maximum(
        jnp.dot(h, w2[...], preferred_element_type=jnp.float32, precision=lax.Precision.HIGHEST) + b2[...],
        0.0)
    h = h * (g2r[...] * rsq) + e2r[...]
    out_ref[...] = (
        jnp.dot(h, wo[...], preferred_element_type=jnp.float32, precision=lax.Precision.HIGHEST) + bo[...])
  return pl.pallas_call(
      body,
      out_shape=jax.ShapeDtypeStruct((G, 1), jnp.float32),
  )(ps, cnt, w1t, b1r, w2t, b2r, g1, be1, g2, be2, wot, bor)


def _amat(a_flat):
  dm = a_flat.shape[0]
  cidx = jnp.arange(dm)[:, None] // HID
  hidx = jnp.arange(16)[None, :]
  return jnp.where(cidx == hidx, a_flat[:, None], 0.0).astype(jnp.float32)


def _mvec(ts, td):
  b = jnp.max(ts, axis=0) + jnp.max(td, axis=0)
  return jnp.maximum(b, 0.2 * b)


def kernel(x, W0, as0, ad0, b0, W1, as1, ad1, b1, W2, as2, ad2, b2,
           fc1_w, fc1_b, fc2_w, fc2_b, bn1_g, bn1_b, bn2_g, bn2_b,
           out_w, out_b, edge_index, batch):
  f32 = jnp.float32
  sl = jnp.arange(N, dtype=edge_index.dtype)
  src = jnp.concatenate([edge_index[0], sl,
                         jnp.zeros((EPAD - E - N,), edge_index.dtype)])
  dst = jnp.concatenate([edge_index[1], sl,
                         jnp.full((EPAD - E - N,), NPAD - 1,
                                  edge_index.dtype)])
  src2d = src.reshape(NTILES * NB, BLK)
  dst2d = dst.reshape(NTILES * NB, BLK)

  rm = (jnp.arange(16)[:, None]
        == (jnp.arange(IN)[None, :] // HID)).astype(f32)
  r2 = (jnp.arange(16)[:, None]
        == (jnp.arange(16)[None, :] // HID)).astype(f32)

  xpad = jnp.pad(x, ((0, NPAD - N), (0, 0)))

  xl0, ts0, td0 = _t0(xpad, W0.T, _amat(as0.reshape(-1)),
                      _amat(ad0.reshape(-1)))
  acc0, den0 = _edge128(xl0, ts0, td0, _mvec(ts0, td0), src2d, dst2d)

  xl1, ts1, td1 = _tmid(acc0, den0, b0.reshape(1, -1), rm, W1.T,
                        _amat(as1.reshape(-1)), _amat(ad1.reshape(-1)), IN)
  acc1, den1 = _edge128(xl1, ts1, td1, _mvec(ts1, td1), src2d, dst2d)

  xl2, ts2, td2 = _tmid(acc1, den1, b1.reshape(1, -1), rm, W2.T,
                        _amat(as2.reshape(-1)), _amat(ad2.reshape(-1)), HID)
  acc2, den2 = _edge16(xl2, ts2, td2, _mvec(ts2, td2), src2d, dst2d)

  batch_row = jnp.pad(batch, (0, NPAD - N),
                      constant_values=G).reshape(NPAD, 1).astype(jnp.int32)
  ps, cnt = _t3(acc2, den2, b2.reshape(1, -1), r2, batch_row)

  return _t4(ps, cnt, fc1_w.T, fc1_b.reshape(1, -1), fc2_w.T,
             fc2_b.reshape(1, -1), bn1_g.reshape(1, -1),
             bn1_b.reshape(1, -1), bn2_g.reshape(1, -1),
             bn2_b.reshape(1, -1), out_w.T, out_b.reshape(1, -1))
